# baseline (speedup 1.0000x reference)
import functools

import jax
import jax.numpy as jnp
from jax import lax
from jax.experimental import pallas as pl
from jax.experimental.pallas import tpu as pltpu
from jax.experimental.pallas import tpu_sc as plsc

V, D, B = 1000000, 64, 16384
NC, NS = 2, 16
NW = NC * NS
B_PER_W = B // NW
CHUNK = 128
NCH = B_PER_W // CHUNK

_mesh = plsc.VectorSubcoreMesh(core_axis_name="c", subcore_axis_name="s")


@functools.partial(
    pl.kernel,
    mesh=_mesh,
    out_type=jax.ShapeDtypeStruct((B, D), jnp.float32),
    scratch_types=[
        pltpu.VMEM((NCH, CHUNK), jnp.int32),
        pltpu.VMEM((B_PER_W, D), jnp.float32),
        pltpu.SemaphoreType.DMA,
    ],
    compiler_params=pltpu.CompilerParams(use_tc_tiling_on_sc=False),
)
def _gather_sc(table_hbm, idx_hbm, out_hbm, idx_v, rows_v, sem):
    wid = lax.axis_index("s") * NC + lax.axis_index("c")
    pltpu.sync_copy(idx_hbm.at[wid], idx_v)
    copies = [
        pltpu.async_copy(
            table_hbm.at[idx_v.at[j]],
            rows_v.at[pl.ds(j * CHUNK, CHUNK)],
            sem,
        )
        for j in range(NCH)
    ]
    for cp in copies:
        cp.wait()
    pltpu.sync_copy(rows_v, out_hbm.at[pl.ds(wid * B_PER_W, B_PER_W)])


def kernel(input, indices):
    idx = indices.astype(jnp.int32).reshape(NW, NCH, CHUNK)
    return _gather_sc(input, idx)


# speedup vs baseline: 1.2874x; 1.2874x over previous
import functools

import jax
import jax.numpy as jnp
from jax import lax
from jax.experimental import pallas as pl
from jax.experimental.pallas import tpu as pltpu
from jax.experimental.pallas import tpu_sc as plsc

V, D, B = 1000000, 64, 16384
NC, NS = 2, 16
NW = NC * NS
B_PER_W = B // NW

_mesh = plsc.VectorSubcoreMesh(core_axis_name="c", subcore_axis_name="s")


@functools.partial(
    pl.kernel,
    mesh=_mesh,
    out_type=jax.ShapeDtypeStruct((B, D), jnp.float32),
    scratch_types=[
        pltpu.VMEM((B_PER_W,), jnp.int32),
        pltpu.VMEM((B_PER_W,), jnp.int32),
        pltpu.SemaphoreType.DMA,
    ],
)
def _gather_sc(table_hbm, bidx_hbm, sidx_hbm, out_hbm, bidx_v, sidx_v, sem):
    wid = lax.axis_index("s") * NC + lax.axis_index("c")
    base = wid * B_PER_W
    pltpu.sync_copy(bidx_hbm.at[wid], bidx_v)
    pltpu.sync_copy(sidx_hbm.at[wid], sidx_v)

    @pl.loop(0, B_PER_W // 16)
    def _(g):
        b_vec = bidx_v[pl.ds(g * 16, 16)]
        s_vec = sidx_v[pl.ds(g * 16, 16)]
        for j in range(16):
            pltpu.async_copy(
                table_hbm.at[b_vec[j], s_vec[j]],
                out_hbm.at[base + g * 16 + j],
                sem,
            )

    pltpu.make_async_copy(
        out_hbm.at[pl.ds(base, B_PER_W)],
        out_hbm.at[pl.ds(base, B_PER_W)],
        sem,
    ).wait()


def kernel(input, indices):
    idx = indices.astype(jnp.int32)
    table3 = input.reshape(V // 8, 8, D)
    bidx = (idx >> 3).reshape(NW, B_PER_W)
    sidx = (idx & 7).reshape(NW, B_PER_W)
    return _gather_sc(table3, bidx, sidx)
